# baseline (speedup 1.0000x reference)
import functools

import jax
import jax.numpy as jnp
from jax import lax
from jax.experimental import pallas as pl
from jax.experimental.pallas import tpu as pltpu
from jax.experimental.pallas import tpu_sc as plsc

MAX_LEN = 8192
HIDDEN = 2048
BATCH = 4
T_LEN = 4096
B_TOTAL = BATCH * T_LEN

_NC = 2
_NS = 16
_NW = _NC * _NS
_BPW = B_TOTAL // _NW
_WINDOW = 64


def _make_gather():
    mesh = plsc.VectorSubcoreMesh(core_axis_name="c", subcore_axis_name="s")

    @functools.partial(
        pl.kernel,
        mesh=mesh,
        compiler_params=pltpu.CompilerParams(needs_layout_passes=False),
        out_type=jax.ShapeDtypeStruct((B_TOTAL, HIDDEN), jnp.float32),
        scratch_types=[
            pltpu.VMEM((_BPW,), jnp.int32),
            pltpu.SemaphoreType.DMA,
        ],
    )
    def gather_kernel(x_hbm, table_hbm, out_hbm, idx_v, sem):
        wid = lax.axis_index("s") * _NC + lax.axis_index("c")
        base = wid * _BPW
        pltpu.sync_copy(
            x_hbm.at[wid >> 3].at[pl.ds((wid & 7) * _BPW, _BPW)], idx_v
        )
        lane = lax.iota(jnp.int32, 16)

        def wait_one(i, c2):
            pltpu.make_async_copy(
                table_hbm.at[pl.ds(0, 1)], out_hbm.at[pl.ds(0, 1)], sem
            ).wait()
            return c2

        def one(i, c2):
            v16 = idx_v[pl.ds((i // 16) * 16, 16)]
            e = jnp.max(jnp.where(lane == (i % 16), v16, 0))
            pltpu.async_copy(
                table_hbm.at[pl.ds(e, 1)], out_hbm.at[pl.ds(base + i, 1)], sem
            )
            return c2

        def body(i, c2):
            one(i, c2)
            return lax.cond(i >= _WINDOW, lambda: wait_one(i, c2), lambda: c2)

        lax.fori_loop(0, _BPW, body, 0)
        lax.fori_loop(0, _WINDOW, wait_one, 0)

    return gather_kernel


_gather = _make_gather()


def kernel(x, weight):
    batch_size, t_length = x.shape
    out = _gather(x.astype(jnp.int32), weight)
    return out.reshape(batch_size, t_length, HIDDEN)


# speedup vs baseline: 35.4773x; 35.4773x over previous
import functools

import jax
import jax.numpy as jnp
from jax import lax
from jax.experimental import pallas as pl
from jax.experimental.pallas import tpu as pltpu
from jax.experimental.pallas import tpu_sc as plsc

MAX_LEN = 8192
HIDDEN = 2048
BATCH = 4
T_LEN = 4096
B_TOTAL = BATCH * T_LEN

_NC = 2
_NS = 16
_NW = _NC * _NS
_BPW = B_TOTAL // _NW
_C = 16
_NB = 2
_NCH = _BPW // _C


def _make_gather():
    mesh = plsc.VectorSubcoreMesh(core_axis_name="c", subcore_axis_name="s")

    @functools.partial(
        pl.kernel,
        mesh=mesh,
        out_type=jax.ShapeDtypeStruct((B_TOTAL, HIDDEN), jnp.float32),
        scratch_types=[
            pltpu.VMEM((_BPW,), jnp.int32),
            pltpu.VMEM((_NB, _C, HIDDEN), jnp.float32),
        ]
        + [pltpu.SemaphoreType.DMA] * (2 * _NB),
    )
    def gather_kernel(x_hbm, table_hbm, out_hbm, idx_v, rows_v, *sems):
        gsems = sems[:_NB]
        ssems = sems[_NB:]
        wid = lax.axis_index("s") * _NC + lax.axis_index("c")
        base = wid * _BPW
        pltpu.sync_copy(
            x_hbm.at[wid >> 3].at[pl.ds((wid & 7) * _BPW, _BPW)], idx_v
        )

        def g_src(g):
            return table_hbm.at[idx_v.at[pl.ds(g * _C, _C)]]

        def o_dst(g):
            return out_hbm.at[pl.ds(base + g * _C, _C)]

        def wait_gather(g, b):
            pltpu.make_async_copy(g_src(g), rows_v.at[b], gsems[b]).wait()

        def wait_out(g, b):
            pltpu.make_async_copy(rows_v.at[b], o_dst(g), ssems[b]).wait()

        for b in range(_NB - 1):
            pltpu.async_copy(g_src(b), rows_v.at[b], gsems[b])

        for b in range(_NB):
            g = b
            h = g + _NB - 1
            if g < _NB - 1:
                wait_gather(g, b)
                pltpu.async_copy(rows_v.at[b], o_dst(g), ssems[b])
            bh = h % _NB
            if h >= _NB:
                wait_out(h - _NB, bh)
            pltpu.async_copy(g_src(h), rows_v.at[bh], gsems[bh])
            if g == _NB - 1:
                wait_gather(g, b)
                pltpu.async_copy(rows_v.at[b], o_dst(g), ssems[b])

        def outer(j, carry):
            for b in range(_NB):
                g = j * _NB + b
                wait_gather(g, b)
                pltpu.async_copy(rows_v.at[b], o_dst(g), ssems[b])
                h = g + _NB - 1
                bh = (b + _NB - 1) % _NB
                wait_out(h - _NB, bh)
                pltpu.async_copy(g_src(h), rows_v.at[bh], gsems[bh])
            return carry

        lax.fori_loop(1, _NCH // _NB - 1, outer, 0)

        for b in range(_NB):
            g = _NCH - _NB + b
            wait_gather(g, b)
            pltpu.async_copy(rows_v.at[b], o_dst(g), ssems[b])
            if b == 0:
                h = _NCH - 1
                bh = h % _NB
                wait_out(h - _NB, bh)
                pltpu.async_copy(g_src(h), rows_v.at[bh], gsems[bh])
        for b in range(_NB):
            g = _NCH - _NB + b
            wait_out(g, b)

    return gather_kernel


_gather = _make_gather()


def kernel(x, weight):
    batch_size, t_length = x.shape
    out = _gather(x.astype(jnp.int32), weight)
    return out.reshape(batch_size, t_length, HIDDEN)


# speedup vs baseline: 36.3642x; 1.0250x over previous
import functools

import jax
import jax.numpy as jnp
from jax import lax
from jax.experimental import pallas as pl
from jax.experimental.pallas import tpu as pltpu
from jax.experimental.pallas import tpu_sc as plsc

MAX_LEN = 8192
HIDDEN = 2048
BATCH = 4
T_LEN = 4096
B_TOTAL = BATCH * T_LEN

_NC = 2
_NS = 16
_NW = _NC * _NS
_BPW = B_TOTAL // _NW
_C = 8
_NB = 4
_NCH = _BPW // _C


def _make_gather():
    mesh = plsc.VectorSubcoreMesh(core_axis_name="c", subcore_axis_name="s")

    @functools.partial(
        pl.kernel,
        mesh=mesh,
        out_type=jax.ShapeDtypeStruct((B_TOTAL, HIDDEN), jnp.float32),
        scratch_types=[
            pltpu.VMEM((_BPW,), jnp.int32),
            pltpu.VMEM((_NB, _C, HIDDEN), jnp.float32),
        ]
        + [pltpu.SemaphoreType.DMA] * (2 * _NB),
    )
    def gather_kernel(x_hbm, table_hbm, out_hbm, idx_v, rows_v, *sems):
        gsems = sems[:_NB]
        ssems = sems[_NB:]
        wid = lax.axis_index("s") * _NC + lax.axis_index("c")
        base = wid * _BPW
        pltpu.sync_copy(
            x_hbm.at[wid >> 3].at[pl.ds((wid & 7) * _BPW, _BPW)], idx_v
        )

        def g_src(g):
            return table_hbm.at[idx_v.at[pl.ds(g * _C, _C)]]

        def o_dst(g):
            return out_hbm.at[pl.ds(base + g * _C, _C)]

        def wait_gather(g, b):
            pltpu.make_async_copy(g_src(g), rows_v.at[b], gsems[b]).wait()

        def wait_out(g, b):
            pltpu.make_async_copy(rows_v.at[b], o_dst(g), ssems[b]).wait()

        for b in range(_NB - 1):
            pltpu.async_copy(g_src(b), rows_v.at[b], gsems[b])

        for b in range(_NB):
            g = b
            h = g + _NB - 1
            if g < _NB - 1:
                wait_gather(g, b)
                pltpu.async_copy(rows_v.at[b], o_dst(g), ssems[b])
            bh = h % _NB
            if h >= _NB:
                wait_out(h - _NB, bh)
            pltpu.async_copy(g_src(h), rows_v.at[bh], gsems[bh])
            if g == _NB - 1:
                wait_gather(g, b)
                pltpu.async_copy(rows_v.at[b], o_dst(g), ssems[b])

        def outer(j, carry):
            for b in range(_NB):
                g = j * _NB + b
                wait_gather(g, b)
                pltpu.async_copy(rows_v.at[b], o_dst(g), ssems[b])
                h = g + _NB - 1
                bh = (b + _NB - 1) % _NB
                wait_out(h - _NB, bh)
                pltpu.async_copy(g_src(h), rows_v.at[bh], gsems[bh])
            return carry

        lax.fori_loop(1, _NCH // _NB - 1, outer, 0)

        for b in range(_NB):
            g = _NCH - _NB + b
            wait_gather(g, b)
            pltpu.async_copy(rows_v.at[b], o_dst(g), ssems[b])
            if b == 0:
                h = _NCH - 1
                bh = h % _NB
                wait_out(h - _NB, bh)
                pltpu.async_copy(g_src(h), rows_v.at[bh], gsems[bh])
        for b in range(_NB):
            g = _NCH - _NB + b
            wait_out(g, b)

    return gather_kernel


_gather = _make_gather()


def kernel(x, weight):
    batch_size, t_length = x.shape
    out = _gather(x.astype(jnp.int32), weight)
    return out.reshape(batch_size, t_length, HIDDEN)
